# Initial kernel scaffold; baseline (speedup 1.0000x reference)
import jax
import jax.numpy as jnp
from jax.experimental import pallas as pl


def kernel(hidden_states, W_router, W1, b1, W2, b2):
    raise NotImplementedError("write your pallas kernel here")


# speedup vs baseline: 3.9843x; 3.9843x over previous
import functools

import jax
import jax.numpy as jnp
from jax import lax
from jax.experimental import pallas as pl
from jax.experimental.pallas import tpu as pltpu
from jax.experimental.pallas import tpu_sc as plsc

HIDDEN = 2048
E = 16
TOKENS = 8192
TB = 1024
NBLK = TOKENS // TB
TPB = TOKENS // 16
ONE_F32_BITS = 0x3F800001


def _tc_body(h_ref, wr_ref, w1_ref, b1_ref, w2_ref, b2_ref,
             logits_ref, probsT_ref, caps_ref, ent_ref,
             acc_ref, entacc_ref):
    i = pl.program_id(0)

    @pl.when(i == 0)
    def _init():
        acc_ref[...] = jnp.zeros_like(acc_ref)
        entacc_ref[0] = 0.0

    h = h_ref[...]
    logits = lax.dot_general(h, wr_ref[...], (((1,), (1,)), ((), ())),
                             preferred_element_type=jnp.float32)
    logits_ref[...] = logits
    m = jnp.max(logits, axis=1, keepdims=True)
    ex = jnp.exp(logits - m)
    p = ex / jnp.sum(ex, axis=1, keepdims=True)
    probsT_ref[...] = p.T

    ent_tok = -jnp.sum(p * jnp.log(p + 1e-8), axis=1)
    entacc_ref[0] += jnp.sum(ent_tok)
    acc_ref[...] += jnp.sum(h, axis=0, keepdims=True)

    @pl.when(i == NBLK - 1)
    def _finish():
        mean = acc_ref[...] * jnp.float32(1.0 / TOKENS)
        h1 = lax.dot_general(mean, w1_ref[...], (((1,), (1,)), ((), ())),
                             preferred_element_type=jnp.float32)
        h1 = jnp.maximum(h1 + b1_ref[...], 0.0)
        cl = lax.dot_general(h1, w2_ref[...], (((1,), (1,)), ((), ())),
                             preferred_element_type=jnp.float32)
        cl = cl + b2_ref[...]
        cm = jnp.max(cl, axis=1, keepdims=True)
        cex = jnp.exp(cl - cm)
        cw = cex / jnp.sum(cex, axis=1, keepdims=True)
        cf = jnp.clip(1.25 + (cw - 0.5) * 1.0, 1.0, 2.0)
        caps_ref[...] = jnp.floor(cf * (TOKENS / E)).astype(jnp.int32)
        ent_ref[...] = (entacc_ref[0] * jnp.float32(1.0 / TOKENS)).reshape(1, 1)


_tc_call = pl.pallas_call(
    _tc_body,
    grid=(NBLK,),
    in_specs=[
        pl.BlockSpec((TB, HIDDEN), lambda i: (i, 0)),
        pl.BlockSpec((E, HIDDEN), lambda i: (0, 0)),
        pl.BlockSpec((HIDDEN // 4, HIDDEN), lambda i: (0, 0)),
        pl.BlockSpec((1, HIDDEN // 4), lambda i: (0, 0)),
        pl.BlockSpec((E, HIDDEN // 4), lambda i: (0, 0)),
        pl.BlockSpec((1, E), lambda i: (0, 0)),
    ],
    out_specs=[
        pl.BlockSpec((TB, E), lambda i: (i, 0)),
        pl.BlockSpec((E, TB), lambda i: (0, i)),
        pl.BlockSpec((1, E), lambda i: (0, 0)),
        pl.BlockSpec((1, 1), lambda i: (0, 0)),
    ],
    out_shape=[
        jax.ShapeDtypeStruct((TOKENS, E), jnp.float32),
        jax.ShapeDtypeStruct((E, TOKENS), jnp.float32),
        jax.ShapeDtypeStruct((1, E), jnp.int32),
        jax.ShapeDtypeStruct((1, 1), jnp.float32),
    ],
    scratch_shapes=[
        pltpu.VMEM((1, HIDDEN), jnp.float32),
        pltpu.SMEM((1,), jnp.float32),
    ],
)


def _splat_i32(x):
    return jnp.full((16,), 1, jnp.int32) * x


@functools.partial(
    pl.kernel,
    out_type=[jax.ShapeDtypeStruct((TOKENS,), jnp.int32),
              jax.ShapeDtypeStruct((TOKENS,), jnp.float32)],
    mesh=plsc.VectorSubcoreMesh(core_axis_name="c", subcore_axis_name="s"),
    compiler_params=pltpu.CompilerParams(needs_layout_passes=False),
    scratch_types=[
        pltpu.VMEM((TOKENS,), jnp.float32),
        pltpu.VMEM((TOKENS,), jnp.int32),
        pltpu.VMEM((16,), jnp.int32),
        pltpu.VMEM((E, TPB), jnp.int32),
        pltpu.VMEM((E, TPB), jnp.float32),
        pltpu.VMEM((TPB,), jnp.int32),
        pltpu.VMEM((TPB,), jnp.float32),
        pltpu.VMEM_SHARED((E, TOKENS), jnp.int32),
    ],
)
def _sc_select(probsT_hbm, caps_hbm, sel_hbm, w_hbm,
               col_v, mask_v, caps_v, mb_v, pb_v, sel_v, w_v, sh_mask):
    c = lax.axis_index("c")
    s = lax.axis_index("s")
    iota16 = lax.broadcasted_iota(jnp.int32, (16,), 0)

    @pl.when(c == 0)
    def _phase_a():
        e = s
        pltpu.sync_copy(probsT_hbm.at[e], col_v)
        pltpu.sync_copy(caps_hbm, caps_v)
        k = jnp.sum(jnp.where(iota16 == e, caps_v[...],
                              jnp.zeros((16,), jnp.int32)))

        def cnt_ge(th):
            thv = _splat_i32(th)

            def body(j, acc):
                bb = plsc.bitcast(col_v[pl.ds(j * 16, 16)], jnp.int32)
                return acc + (bb >= thv).astype(jnp.int32)

            acc = lax.fori_loop(0, TOKENS // 16, body,
                                jnp.zeros((16,), jnp.int32), unroll=8)
            return jnp.sum(acc)

        def bs_body(t, lohi):
            lo, hi = lohi
            mid = (lo + hi) >> 1
            big = cnt_ge(mid) >= k
            return (jnp.where(big, mid, lo), jnp.where(big, hi, mid))

        lo, _ = lax.fori_loop(0, 30, bs_body,
                              (jnp.int32(0), jnp.int32(ONE_F32_BITS)))
        thr = lo
        need = k - cnt_ge(thr + 1)
        thrv = _splat_i32(thr)
        needv = _splat_i32(need)

        def fin_body(j, eqcnt):
            bb = plsc.bitcast(col_v[pl.ds(j * 16, 16)], jnp.int32)
            gt = bb > thrv
            eq = bb == thrv
            eqi = eq.astype(jnp.int32)
            rank = plsc.cumsum(eqi) - eqi + eqcnt
            take = jnp.logical_and(eq, rank < needv)
            mask_v[pl.ds(j * 16, 16)] = jnp.logical_or(gt, take).astype(jnp.int32)
            return eqcnt + jnp.sum(eqi)

        lax.fori_loop(0, TOKENS // 16, fin_body, jnp.int32(0), unroll=4)
        pltpu.sync_copy(mask_v, sh_mask.at[e])

    plsc.subcore_barrier()

    @pl.when(c == 0)
    def _phase_b():
        base = s * TPB
        for ee in range(E):
            pltpu.sync_copy(sh_mask.at[ee, pl.ds(base, TPB)], mb_v.at[ee])
            pltpu.sync_copy(probsT_hbm.at[ee, pl.ds(base, TPB)], pb_v.at[ee])

        def body(j, carry):
            best = jnp.zeros((16,), jnp.int32)
            w = jnp.zeros((16,), jnp.float32)
            for ee in range(E):
                selb = mb_v[ee, pl.ds(j * 16, 16)] > 0
                best = jnp.where(selb, jnp.full((16,), ee, jnp.int32), best)
                w = jnp.where(selb, pb_v[ee, pl.ds(j * 16, 16)], w)
            sel_v[pl.ds(j * 16, 16)] = best
            w_v[pl.ds(j * 16, 16)] = w
            return carry

        lax.fori_loop(0, TPB // 16, body, jnp.int32(0), unroll=2)
        pltpu.sync_copy(sel_v, sel_hbm.at[pl.ds(base, TPB)])
        pltpu.sync_copy(w_v, w_hbm.at[pl.ds(base, TPB)])


def kernel(hidden_states, W_router, W1, b1, W2, b2):
    logits, probsT, caps2d, ent2d = _tc_call(
        hidden_states, W_router, W1, b1.reshape(1, -1), W2, b2.reshape(1, -1))
    sel, w = _sc_select(probsT, caps2d.reshape(E))
    return logits, sel[:, None], w[:, None], ent2d.reshape(())
